# baseline (speedup 1.0000x reference)
import functools

import jax
import jax.numpy as jnp
from jax import lax
from jax.experimental import pallas as pl
from jax.experimental.pallas import tpu as pltpu
from jax.experimental.pallas import tpu_sc as plsc

N = 10000
E = 320000
D = 128

NC = 2
NS = 16
NW = NC * NS

CHUNK = 80
NCHUNK = 128
EPAD = NW * NCHUNK * CHUNK
NPAD = 10240
RPT = NPAD // NS
ZROWS = CHUNK


def _sc_aggregate(x, src, dst):
    mesh = plsc.VectorSubcoreMesh(core_axis_name="c", subcore_axis_name="s")

    @functools.partial(
        pl.kernel,
        mesh=mesh,
        out_type=jax.ShapeDtypeStruct((NC, NPAD, D), jnp.float32),
        scratch_types=[
            pltpu.VMEM((NCHUNK, CHUNK), jnp.int32),
            pltpu.VMEM((NCHUNK, CHUNK), jnp.int32),
            pltpu.VMEM((CHUNK, D), jnp.float32),
            pltpu.VMEM_SHARED((NPAD, D), jnp.float32),
        ],
    )
    def agg_kernel(x_hbm, src_hbm, dst_hbm, out_hbm,
                   sidx_v, didx_v, rows_v, acc):
        c = lax.axis_index("c")
        s = lax.axis_index("s")
        wid = s * NC + c

        def zero_row(r, carry):
            for c0 in range(0, D, 16):
                rows_v[r, pl.ds(c0, 16)] = jnp.zeros((16,), jnp.float32)
            return carry
        lax.fori_loop(0, ZROWS, zero_row, 0)
        for t in range(RPT // ZROWS):
            pltpu.sync_copy(rows_v, acc.at[pl.ds(s * RPT + t * ZROWS, ZROWS)])
        plsc.subcore_barrier()

        pltpu.sync_copy(src_hbm.at[pl.ds(wid * NCHUNK, NCHUNK)], sidx_v)
        pltpu.sync_copy(dst_hbm.at[pl.ds(wid * NCHUNK, NCHUNK)], didx_v)

        def body(j, carry):
            pltpu.sync_copy(x_hbm.at[sidx_v.at[j]], rows_v)
            pltpu.sync_copy(rows_v, acc.at[didx_v.at[j]], add=True)
            return carry
        lax.fori_loop(0, NCHUNK, body, 0)
        plsc.subcore_barrier()

        for t in range(RPT // ZROWS):
            r0 = s * RPT + t * ZROWS
            pltpu.sync_copy(acc.at[pl.ds(r0, ZROWS)],
                            out_hbm.at[c, pl.ds(r0, ZROWS)])

    return agg_kernel(x, src, dst)


def _mlp_kernel(x_ref, p_ref, w1_ref, b1_ref, g_ref, be_ref, w2_ref, b2_ref,
                o_ref):
    h = x_ref[...] + p_ref[0, :N, :] + p_ref[1, :N, :]
    z = lax.dot_general(h, w1_ref[...], (((1,), (1,)), ((), ())),
                        preferred_element_type=jnp.float32)
    z = jnp.maximum(z + b1_ref[...], 0.0)
    mean = jnp.mean(z, axis=0, keepdims=True)
    var = jnp.mean(z * z, axis=0, keepdims=True) - mean * mean
    scale = g_ref[...] * lax.rsqrt(var + 1e-5)
    zn = (z - mean) * scale + be_ref[...]
    o_ref[...] = lax.dot_general(zn, w2_ref[...], (((1,), (1,)), ((), ())),
                                 preferred_element_type=jnp.float32) + b2_ref[...]


def _mlp(x, parts, W1, b1, gamma, beta, W2, b2):
    return pl.pallas_call(
        _mlp_kernel,
        out_shape=jax.ShapeDtypeStruct((N, D), jnp.float32),
    )(x, parts, W1, b1.reshape(1, D), gamma.reshape(1, D),
      beta.reshape(1, D), W2, b2.reshape(1, D))


def kernel(x, edge_index, W1, b1, gamma, beta, W2, b2):
    npad_e = EPAD - E
    src = jnp.concatenate(
        [edge_index[0], jnp.zeros((npad_e,), jnp.int32)]).reshape(
            NW * NCHUNK, CHUNK)
    pad_dst = N + (jnp.arange(npad_e, dtype=jnp.int32) % (NPAD - N))
    dst = jnp.concatenate(
        [edge_index[1], pad_dst]).reshape(NW * NCHUNK, CHUNK)
    parts = _sc_aggregate(x, src, dst)
    return _mlp(x, parts, W1, b1, gamma, beta, W2, b2)


# speedup vs baseline: 1.0802x; 1.0802x over previous
import functools

import jax
import jax.numpy as jnp
from jax import lax
from jax.experimental import pallas as pl
from jax.experimental.pallas import tpu as pltpu
from jax.experimental.pallas import tpu_sc as plsc

N = 10000
E = 320000
D = 128

NC = 2
NS = 16
NW = NC * NS

CHUNK = 80
NSLOT = 4
NGROUP = 32
NCHUNK = NSLOT * NGROUP
EPW = NCHUNK * CHUNK
EPAD = NW * EPW

NPAD = 10240
RPT = NPAD // NS


def _sc_aggregate(x, src, dst):
    mesh = plsc.VectorSubcoreMesh(core_axis_name="c", subcore_axis_name="s")

    @functools.partial(
        pl.kernel,
        mesh=mesh,
        out_type=jax.ShapeDtypeStruct((NC, NPAD, D), jnp.float32),
        scratch_types=(
            [pltpu.VMEM((CHUNK,), jnp.int32) for _ in range(NSLOT)]
            + [pltpu.VMEM((CHUNK,), jnp.int32) for _ in range(NSLOT)]
            + [pltpu.VMEM((CHUNK, D), jnp.float32) for _ in range(NSLOT)]
            + [pltpu.VMEM_SHARED((NPAD, D), jnp.float32)]
            + [pltpu.SemaphoreType.DMA for _ in range(3 * NSLOT)]
        ),
    )
    def agg_kernel(x_hbm, src_hbm, dst_hbm, out_hbm, *scr):
        sidx = scr[0:NSLOT]
        didx = scr[NSLOT:2 * NSLOT]
        rows = scr[2 * NSLOT:3 * NSLOT]
        acc = scr[3 * NSLOT]
        isem = scr[3 * NSLOT + 1:3 * NSLOT + 1 + NSLOT]
        gsem = scr[3 * NSLOT + 1 + NSLOT:3 * NSLOT + 1 + 2 * NSLOT]
        ssem = scr[3 * NSLOT + 1 + 2 * NSLOT:3 * NSLOT + 1 + 3 * NSLOT]

        c = lax.axis_index("c")
        s = lax.axis_index("s")
        wid = s * NC + c

        def zero_row(r, carry):
            for c0 in range(0, D, 16):
                rows[0][r, pl.ds(c0, 16)] = jnp.zeros((16,), jnp.float32)
            return carry
        lax.fori_loop(0, CHUNK, zero_row, 0)
        for t in range(RPT // CHUNK):
            pltpu.sync_copy(rows[0], acc.at[pl.ds(s * RPT + t * CHUNK, CHUNK)])
        plsc.subcore_barrier()

        ebase = wid * EPW

        def group(k, carry):
            @pl.when(k > 0)
            def _():
                for t in range(NSLOT):
                    pltpu.make_async_copy(
                        rows[t], acc.at[didx[t]], ssem[t]).wait()
            idescs = []
            for t in range(NSLOT):
                off = ebase + (k * NSLOT + t) * CHUNK
                idescs.append((
                    pltpu.async_copy(src_hbm.at[pl.ds(off, CHUNK)],
                                     sidx[t], isem[t]),
                    pltpu.async_copy(dst_hbm.at[pl.ds(off, CHUNK)],
                                     didx[t], isem[t]),
                ))
            gdescs = []
            for t in range(NSLOT):
                idescs[t][0].wait()
                idescs[t][1].wait()
                gdescs.append(
                    pltpu.async_copy(x_hbm.at[sidx[t]], rows[t], gsem[t]))
            for t in range(NSLOT):
                gdescs[t].wait()
                pltpu.async_copy(rows[t], acc.at[didx[t]], ssem[t], add=True)
            return carry
        lax.fori_loop(0, NGROUP, group, 0)
        for t in range(NSLOT):
            pltpu.make_async_copy(rows[t], acc.at[didx[t]], ssem[t]).wait()
        plsc.subcore_barrier()

        wdescs = []
        for t in range(RPT // CHUNK):
            r0 = s * RPT + t * CHUNK
            wdescs.append(
                pltpu.async_copy(acc.at[pl.ds(r0, CHUNK)],
                                 out_hbm.at[c, pl.ds(r0, CHUNK)],
                                 gsem[t % NSLOT]))
        for d in wdescs:
            d.wait()

    return agg_kernel(x, src, dst)


def _mlp_kernel(x_ref, p_ref, w1_ref, b1_ref, g_ref, be_ref, w2_ref, b2_ref,
                o_ref):
    h = x_ref[...] + p_ref[0, :N, :] + p_ref[1, :N, :]
    z = lax.dot_general(h, w1_ref[...], (((1,), (1,)), ((), ())),
                        preferred_element_type=jnp.float32)
    z = jnp.maximum(z + b1_ref[...], 0.0)
    mean = jnp.mean(z, axis=0, keepdims=True)
    var = jnp.mean(z * z, axis=0, keepdims=True) - mean * mean
    scale = g_ref[...] * lax.rsqrt(var + 1e-5)
    zn = (z - mean) * scale + be_ref[...]
    o_ref[...] = lax.dot_general(zn, w2_ref[...], (((1,), (1,)), ((), ())),
                                 preferred_element_type=jnp.float32) + b2_ref[...]


def _mlp(x, parts, W1, b1, gamma, beta, W2, b2):
    return pl.pallas_call(
        _mlp_kernel,
        out_shape=jax.ShapeDtypeStruct((N, D), jnp.float32),
    )(x, parts, W1, b1.reshape(1, D), gamma.reshape(1, D),
      beta.reshape(1, D), W2, b2.reshape(1, D))


def kernel(x, edge_index, W1, b1, gamma, beta, W2, b2):
    npad_e = EPAD - E
    src = jnp.concatenate([edge_index[0], jnp.zeros((npad_e,), jnp.int32)])
    pad_dst = N + (jnp.arange(npad_e, dtype=jnp.int32) % (NPAD - N))
    dst = jnp.concatenate([edge_index[1], pad_dst])
    parts = _sc_aggregate(x, src, dst)
    return _mlp(x, parts, W1, b1, gamma, beta, W2, b2)
